# baseline (speedup 1.0000x reference)
import functools

import jax
import jax.numpy as jnp
from jax import lax
from jax.experimental import pallas as pl
from jax.experimental.pallas import tpu as pltpu
from jax.experimental.pallas import tpu_sc as plsc

N = 10000
E = 320000
D = 128
NC = 40
NH = 128
NL = 4

ROWS = 400
GRID = N // ROWS

NSC = 2
NSUB = 16
NW = NSC * NSUB
EPW = E // NW
B = 128
EPP = 10240
NBT = EPP // B
G = 8
NG = NBT // G
NA = 10240
RPS = NA // NSUB


def _spmm_body(W, xw, srcs, dsts, zeros, out, srcb0, dstb0, srcb1, dstb1,
               rows0, rows1, acc, gs0, gs1, is0, id0, is1, id1):
    c = lax.axis_index("c")
    s = lax.axis_index("s")
    wid = c * NSUB + s
    row0 = s * RPS
    rows = [rows0, rows1]
    gsem = [gs0, gs1]
    srcb = [srcb0, srcb1]
    dstb = [dstb0, dstb1]
    isem = [is0, is1]
    idem = [id0, id1]
    pltpu.sync_copy(zeros.at[pl.ds(row0, RPS)], acc.at[pl.ds(row0, RPS)])
    pltpu.sync_copy(srcs.at[wid, pl.ds(0, G)], srcb[0])
    pltpu.sync_copy(dsts.at[wid, pl.ds(0, G)], dstb[0])
    pltpu.async_copy(srcs.at[wid, pl.ds(G, G)], srcb[1], isem[1])
    pltpu.async_copy(dsts.at[wid, pl.ds(G, G)], dstb[1], idem[1])
    plsc.subcore_barrier()
    pltpu.async_copy(xw.at[srcb[0].at[0]], rows[0], gsem[0])

    def pair(g2, carry):
        for pp in range(2):
            g = g2 * 2 + pp
            sb, db = srcb[pp], dstb[pp]
            ob, odb = srcb[1 - pp], dstb[1 - pp]
            for i in range(G):
                j = g * G + i
                rb = i % 2
                nrb = (i + 1) % 2
                pltpu.make_async_copy(xw.at[sb.at[i]], rows[rb],
                                      gsem[rb]).wait()
                if i < G - 1:
                    pltpu.async_copy(xw.at[sb.at[i + 1]], rows[nrb],
                                     gsem[nrb])
                else:
                    @pl.when(g < NG - 1)
                    def _cross_group():
                        pltpu.make_async_copy(
                            srcs.at[wid, pl.ds((g + 1) * G, G)], ob,
                            isem[1 - pp]).wait()
                        pltpu.make_async_copy(
                            dsts.at[wid, pl.ds((g + 1) * G, G)], odb,
                            idem[1 - pp]).wait()
                        pltpu.async_copy(xw.at[ob.at[0]], rows[nrb],
                                         gsem[nrb])
                pltpu.sync_copy(rows[rb], acc.at[db.at[i]], add=True)

            @pl.when(g < NG - 2)
            def _prefetch_idx():
                pltpu.async_copy(srcs.at[wid, pl.ds((g + 2) * G, G)], sb,
                                 isem[pp])
                pltpu.async_copy(dsts.at[wid, pl.ds((g + 2) * G, G)], db,
                                 idem[pp])
        return carry

    lax.fori_loop(0, NG // 2, pair, 0)
    plsc.subcore_barrier()
    pltpu.sync_copy(acc.at[pl.ds(row0, RPS)], out.at[c, pl.ds(row0, RPS)])


@functools.partial(jax.jit, static_argnames=("W",))
def _spmm(xw, srcs, dsts, zeros, W):
    mesh = plsc.VectorSubcoreMesh(core_axis_name="c", subcore_axis_name="s")
    body = functools.partial(_spmm_body, W)
    return pl.kernel(
        body,
        out_type=jax.ShapeDtypeStruct((NSC, NA, W), jnp.float32),
        mesh=mesh,
        scratch_types=[
            pltpu.VMEM((G, B), jnp.int32),
            pltpu.VMEM((G, B), jnp.int32),
            pltpu.VMEM((G, B), jnp.int32),
            pltpu.VMEM((G, B), jnp.int32),
            pltpu.VMEM((B, W), jnp.float32),
            pltpu.VMEM((B, W), jnp.float32),
            pltpu.VMEM_SHARED((NA, W), jnp.float32),
            pltpu.SemaphoreType.DMA,
            pltpu.SemaphoreType.DMA,
            pltpu.SemaphoreType.DMA,
            pltpu.SemaphoreType.DMA,
            pltpu.SemaphoreType.DMA,
            pltpu.SemaphoreType.DMA,
        ],
        name=f"sc_spmm_{W}",
    )(xw, srcs, dsts, zeros)


def _xc_body(x_ref, idxl_ref, labsel_ref, o_ref):
    i = pl.program_id(0)
    rid = lax.broadcasted_iota(jnp.int32, (ROWS, 1000), 0) + i * ROWS
    m1 = (rid == idxl_ref[...][None, :]).astype(jnp.float32)
    cid = lax.broadcasted_iota(jnp.int32, (1000, NC), 1)
    m2 = (labsel_ref[...][:, None] == cid).astype(jnp.float32)
    feats = jnp.minimum(
        jnp.dot(m1, m2, preferred_element_type=jnp.float32), 1.0)
    o_ref[:, :D] = x_ref[...]
    o_ref[:, D:] = feats


def _build_xc(x, idx_labeled, lab_sel):
    return pl.pallas_call(
        _xc_body,
        out_shape=jax.ShapeDtypeStruct((N, D + NC), jnp.float32),
        grid=(GRID,),
        in_specs=[
            pl.BlockSpec((ROWS, D), lambda i: (i, 0)),
            pl.BlockSpec((1000,), lambda i: (0,)),
            pl.BlockSpec((1000,), lambda i: (0,)),
        ],
        out_specs=pl.BlockSpec((ROWS, D + NC), lambda i: (i, 0)),
        name="tc_xc",
    )(x, idx_labeled, lab_sel)


def _mm_body(x_ref, w_ref, o_ref):
    o_ref[...] = jnp.dot(x_ref[...], w_ref[...],
                         preferred_element_type=jnp.float32)


def _mm(x, w):
    K = x.shape[1]
    M = w.shape[1]
    return pl.pallas_call(
        _mm_body,
        out_shape=jax.ShapeDtypeStruct((N, M), jnp.float32),
        grid=(GRID,),
        in_specs=[
            pl.BlockSpec((ROWS, K), lambda i: (i, 0)),
            pl.BlockSpec((K, M), lambda i: (0, 0)),
        ],
        out_specs=pl.BlockSpec((ROWS, M), lambda i: (i, 0)),
        name="tc_mm",
    )(x, w)


def _act_body(s_ref, b_ref, o_ref):
    o_ref[...] = jnp.tanh(s_ref[0] + s_ref[1] + b_ref[...][None, :])


def _act(parts, b):
    return pl.pallas_call(
        _act_body,
        out_shape=jax.ShapeDtypeStruct((N, NH), jnp.float32),
        grid=(GRID,),
        in_specs=[
            pl.BlockSpec((NSC, ROWS, NH), lambda i: (0, i, 0)),
            pl.BlockSpec((NH,), lambda i: (0,)),
        ],
        out_specs=pl.BlockSpec((ROWS, NH), lambda i: (i, 0)),
        name="tc_act",
    )(parts, b)


def _lsm_body(s_ref, b_ref, o_ref):
    t = s_ref[0] + s_ref[1] + b_ref[...][None, :]
    mask = lax.broadcasted_iota(jnp.int32, (ROWS, NH), 1) < NC
    t = jnp.where(mask, t, -jnp.inf)
    m = jnp.max(t, axis=1, keepdims=True)
    e = jnp.where(mask, jnp.exp(t - m), 0.0)
    lse = jnp.log(jnp.sum(e, axis=1, keepdims=True))
    o_ref[...] = (t - m - lse)[:, :NC]


def _lsm(parts, b_pad):
    return pl.pallas_call(
        _lsm_body,
        out_shape=jax.ShapeDtypeStruct((N, NC), jnp.float32),
        grid=(GRID,),
        in_specs=[
            pl.BlockSpec((NSC, ROWS, NH), lambda i: (0, i, 0)),
            pl.BlockSpec((NH,), lambda i: (0,)),
        ],
        out_specs=pl.BlockSpec((ROWS, NC), lambda i: (i, 0)),
        name="tc_lsm",
    )(parts, b_pad)


def kernel(x, W0, b0, W1, b1, W2, b2, W3, b3, W_out, b_out, edge_index,
           labels, idx_labeled):
    Ws = [W0, W1, W2, W3]
    bs = [b0, b1, b2, b3]
    srcs = jnp.pad(edge_index[0].reshape(NW, EPW), ((0, 0), (0, EPP - EPW)),
                   constant_values=0).reshape(NW, NBT, B)
    dsts = jnp.pad(edge_index[1].reshape(NW, EPW), ((0, 0), (0, EPP - EPW)),
                   constant_values=N).reshape(NW, NBT, B)
    lab_sel = jnp.take(labels, idx_labeled)
    zeros128 = jnp.zeros((NA, NH), jnp.float32)

    xc = _build_xc(x, idx_labeled, lab_sel)
    blocks = []
    for k in range(NL):
        inp = jnp.concatenate([xc] + blocks, axis=1) if blocks else xc
        xw = _mm(inp, Ws[k])
        parts = _spmm(xw, srcs, dsts, zeros128, W=NH)
        blocks.append(_act(parts, bs[k]))
    inp = jnp.concatenate([xc] + blocks, axis=1)
    w_pad = jnp.pad(W_out, ((0, 0), (0, NH - NC)))
    z = _mm(inp, w_pad)
    parts = _spmm(z, srcs, dsts, zeros128, W=NH)
    b_pad = jnp.pad(b_out, (0, NH - NC))
    return _lsm(parts, b_pad)
